# baseline (speedup 1.0000x reference)
import functools

import jax
import jax.numpy as jnp
from jax import lax
from jax.experimental import pallas as pl
from jax.experimental.pallas import tpu as pltpu
from jax.experimental.pallas import tpu_sc as plsc

N_NODES = 10000
N_EDGES = 320000
F = 128
NC, NS = 2, 16
NW = NC * NS
EPW = N_EDGES // NW
CH = 80
NCHUNK = EPW // CH
P_NODES = 10240
RPS = P_NODES // NS


def _sc_aggregate(x, src, dst, w, zeros):
  mesh = plsc.VectorSubcoreMesh(
      core_axis_name="c", subcore_axis_name="s", num_cores=NC, num_subcores=NS)

  NB = 4
  DB = 8
  GL = 2
  IL = 4

  @functools.partial(
      pl.kernel,
      out_type=jax.ShapeDtypeStruct((NC, P_NODES, F), jnp.float32),
      mesh=mesh,
      scratch_types=dict(
          src_v=pltpu.VMEM((NB, CH), jnp.int32),
          dst_v=pltpu.VMEM((DB, CH), jnp.int32),
          w_v=pltpu.VMEM((NB, CH), jnp.float32),
          rows_v=pltpu.VMEM((NB, CH, F // 2), jnp.int32),
          prod_v=pltpu.VMEM((2, CH, F), jnp.float32),
          acc_sh=pltpu.VMEM_SHARED((P_NODES, F), jnp.float32),
          sem_g=pltpu.SemaphoreType.DMA((NB,)),
          sem_i=pltpu.SemaphoreType.DMA((NB,)),
          sem_d=pltpu.SemaphoreType.DMA((DB,)),
          sem_s=pltpu.SemaphoreType.DMA((2,)),
      ),
      compiler_params=pltpu.CompilerParams(needs_layout_passes=False,
                                           use_tc_tiling_on_sc=False),
  )
  def k(x_hbm, src_hbm, dst_hbm, w_hbm, z_hbm, out_hbm,
        src_v, dst_v, w_v, rows_v, prod_v, acc_sh, sem_g, sem_i, sem_d,
        sem_s):
    cid = lax.axis_index("c")
    sid = lax.axis_index("s")
    wid = cid * NS + sid
    base = wid * EPW

    def issue_sw(cc, b):
      off = base + cc * CH
      pltpu.async_copy(src_hbm.at[pl.ds(off, CH)], src_v.at[b], sem_i.at[b])
      pltpu.async_copy(w_hbm.at[pl.ds(off, CH)], w_v.at[b], sem_i.at[b])

    def wait_sw(b):
      pltpu.make_async_copy(src_hbm.at[pl.ds(0, CH)], src_v.at[b],
                            sem_i.at[b]).wait()
      pltpu.make_async_copy(w_hbm.at[pl.ds(0, CH)], w_v.at[b],
                            sem_i.at[b]).wait()

    def issue_d(cc, bd):
      off = base + cc * CH
      pltpu.async_copy(dst_hbm.at[pl.ds(off, CH)], dst_v.at[bd], sem_d.at[bd])

    def wait_d(bd):
      pltpu.make_async_copy(dst_hbm.at[pl.ds(0, CH)], dst_v.at[bd],
                            sem_d.at[bd]).wait()

    def start_gather(b):
      pltpu.async_copy(x_hbm.at[src_v.at[b]], rows_v.at[b], sem_g.at[b])

    def wait_gather(b):
      pltpu.make_async_copy(x_hbm.at[src_v.at[b]], rows_v.at[b],
                            sem_g.at[b]).wait()

    def start_scatter(b2, bd):
      pltpu.async_copy(prod_v.at[b2], acc_sh.at[dst_v.at[bd]], sem_s.at[b2],
                       add=True)

    def wait_scatter(b2):
      pltpu.make_async_copy(prod_v.at[b2], acc_sh.at[dst_v.at[0]],
                            sem_s.at[b2]).wait()

    for cc in range(IL):
      issue_sw(cc, cc % NB)
      issue_d(cc, cc % DB)
    pltpu.sync_copy(z_hbm, acc_sh.at[pl.ds(sid * RPS, RPS)])
    plsc.subcore_barrier()
    for cc in range(GL):
      wait_sw(cc % NB)
      start_gather(cc % NB)

    @pl.loop(0, NCHUNK)
    def _chunk(c):
      b = c % NB
      bd = c % DB

      b2 = c % 2

      wait_gather(b)

      @pl.when(c + GL < NCHUNK)
      def _():
        bg = (c + GL) % NB
        wait_sw(bg)
        start_gather(bg)

      @pl.when(c >= 2)
      def _():
        wait_scatter(b2)

      iota16 = lax.iota(jnp.int32, 16)
      for g in range(CH // 16):
        wv = w_v[b, pl.ds(g * 16, 16)]
        for i in range(16):
          ws = jnp.max(jnp.where(iota16 == i, wv, -jnp.inf))
          wb = jnp.broadcast_to(ws, (16,))
          r = g * 16 + i
          for j in range(F // 32):
            ab = plsc.bitcast(rows_v[b, r, pl.ds(j * 16, 16)], jnp.bfloat16)
            lo, hi = plsc.unpack(ab, format=plsc.PackFormat.INTERLEAVED)
            prod_v[b2, r, pl.ds(j * 32, 16)] = lo * wb
            prod_v[b2, r, pl.ds(j * 32 + 16, 16)] = hi * wb

      wait_d(bd)
      start_scatter(b2, bd)

      @pl.when(c + IL < NCHUNK)
      def _():
        issue_sw(c + IL, (c + IL) % NB)
        issue_d(c + IL, (c + IL) % DB)

    for cc in range(NCHUNK - 2, NCHUNK):
      wait_scatter(cc % 2)
    plsc.subcore_barrier()
    pltpu.sync_copy(acc_sh.at[pl.ds(sid * RPS, RPS)],
                    out_hbm.at[cid, pl.ds(sid * RPS, RPS)])

  return k(x, src, dst, w, zeros)


BLK = 400


def _tc_finish(p0, p1, Wm, b2):
  def body(p0_ref, p1_ref, w_ref, b_ref, o_ref):
    acc = p0_ref[...] + p1_ref[...]
    h = jnp.dot(acc, w_ref[...], preferred_element_type=jnp.float32)
    o_ref[...] = jnp.maximum(h + b_ref[...], 0.0)

  return pl.pallas_call(
      body,
      grid=(N_NODES // BLK,),
      in_specs=[
          pl.BlockSpec((BLK, F), lambda i: (i, 0)),
          pl.BlockSpec((BLK, F), lambda i: (i, 0)),
          pl.BlockSpec((F, F), lambda i: (0, 0)),
          pl.BlockSpec((1, F), lambda i: (0, 0)),
      ],
      out_specs=pl.BlockSpec((BLK, F), lambda i: (i, 0)),
      out_shape=jax.ShapeDtypeStruct((N_NODES, F), jnp.float32),
  )(p0, p1, Wm, b2)


import numpy as _np

_UNPACK_PERM = _np.array(
    [j * 32 + (2 * k if k < 16 else 2 * (k - 16) + 1)
     for j in range(F // 32) for k in range(32)], dtype=_np.int32)


def kernel(x, edge_index, edge_weight, W, b):
  src = edge_index[0].astype(jnp.int32)
  dst = edge_index[1].astype(jnp.int32)
  zeros = jnp.zeros((RPS, F), jnp.float32)
  x_bf = x.astype(jnp.bfloat16)
  x_i32 = lax.bitcast_convert_type(
      x_bf.reshape(N_NODES, F // 2, 2), jnp.int32)
  parts = _sc_aggregate(x_i32, src, dst, edge_weight, zeros)
  w_perm = W[_UNPACK_PERM, :]
  return _tc_finish(parts[0, :N_NODES], parts[1, :N_NODES], w_perm,
                    b[None, :])


# speedup vs baseline: 2.8193x; 2.8193x over previous
import functools

import jax
import jax.numpy as jnp
from jax import lax
from jax.experimental import pallas as pl
from jax.experimental.pallas import tpu as pltpu
from jax.experimental.pallas import tpu_sc as plsc

N_NODES = 10000
N_EDGES = 320000
F = 128
NC, NS = 2, 16
NW = NC * NS
EPW = N_EDGES // NW
CH = 80
NCHUNK = EPW // CH
P_NODES = 10240
RPS = P_NODES // NS


def _sc_aggregate(x, src, dst, w, zeros):
  mesh = plsc.VectorSubcoreMesh(
      core_axis_name="c", subcore_axis_name="s", num_cores=NC, num_subcores=NS)

  NB = 4
  DB = 8
  GL = 2
  IL = 4

  @functools.partial(
      pl.kernel,
      out_type=jax.ShapeDtypeStruct((NC, P_NODES, F), jnp.float32),
      mesh=mesh,
      scratch_types=dict(
          src_v=pltpu.VMEM((NB, CH), jnp.int32),
          dst_v=pltpu.VMEM((DB, CH), jnp.int32),
          w_v=pltpu.VMEM((NB, CH), jnp.float32),
          rows_v=pltpu.VMEM((NB, CH, F), jnp.float32),
          acc_sh=pltpu.VMEM_SHARED((P_NODES, F), jnp.float32),
          sem_g=pltpu.SemaphoreType.DMA((NB,)),
          sem_i=pltpu.SemaphoreType.DMA((NB,)),
          sem_d=pltpu.SemaphoreType.DMA((DB,)),
          sem_s=pltpu.SemaphoreType.DMA((NB,)),
      ),
      compiler_params=pltpu.CompilerParams(needs_layout_passes=False),
  )
  def k(x_hbm, src_hbm, dst_hbm, w_hbm, z_hbm, out_hbm,
        src_v, dst_v, w_v, rows_v, acc_sh, sem_g, sem_i, sem_d, sem_s):
    cid = lax.axis_index("c")
    sid = lax.axis_index("s")
    wid = cid * NS + sid
    base = wid * EPW

    def issue_sw(cc, b):
      off = base + cc * CH
      pltpu.async_copy(src_hbm.at[pl.ds(off, CH)], src_v.at[b], sem_i.at[b])
      pltpu.async_copy(w_hbm.at[pl.ds(off, CH)], w_v.at[b], sem_i.at[b])

    def wait_sw(b):
      pltpu.make_async_copy(src_hbm.at[pl.ds(0, CH)], src_v.at[b],
                            sem_i.at[b]).wait()
      pltpu.make_async_copy(w_hbm.at[pl.ds(0, CH)], w_v.at[b],
                            sem_i.at[b]).wait()

    def issue_d(cc, bd):
      off = base + cc * CH
      pltpu.async_copy(dst_hbm.at[pl.ds(off, CH)], dst_v.at[bd], sem_d.at[bd])

    def wait_d(bd):
      pltpu.make_async_copy(dst_hbm.at[pl.ds(0, CH)], dst_v.at[bd],
                            sem_d.at[bd]).wait()

    def start_gather(b):
      pltpu.async_copy(x_hbm.at[src_v.at[b]], rows_v.at[b], sem_g.at[b])

    def wait_gather(b):
      pltpu.make_async_copy(x_hbm.at[src_v.at[b]], rows_v.at[b],
                            sem_g.at[b]).wait()

    def start_scatter(b, bd):
      pltpu.async_copy(rows_v.at[b], acc_sh.at[dst_v.at[bd]], sem_s.at[b],
                       add=True)

    def wait_scatter(b):
      pltpu.make_async_copy(rows_v.at[b], acc_sh.at[dst_v.at[0]],
                            sem_s.at[b]).wait()

    for cc in range(IL):
      issue_sw(cc, cc % NB)
      issue_d(cc, cc % DB)
    pltpu.sync_copy(z_hbm, acc_sh.at[pl.ds(sid * RPS, RPS)])
    plsc.subcore_barrier()
    for cc in range(GL):
      wait_sw(cc % NB)
      start_gather(cc % NB)

    @pl.loop(0, NCHUNK)
    def _chunk(c):
      b = c % NB
      bd = c % DB

      wait_gather(b)

      @pl.when(c + GL < NCHUNK)
      def _():
        bg = (c + GL) % NB
        wait_sw(bg)

        @pl.when(c + GL >= NB)
        def _():
          wait_scatter(bg)

        start_gather(bg)

      iota16 = lax.iota(jnp.int32, 16)
      for g in range(CH // 16):
        wv = w_v[b, pl.ds(g * 16, 16)]
        for i in range(16):
          ws = jnp.max(jnp.where(iota16 == i, wv, -jnp.inf))
          wb = jnp.broadcast_to(ws, (16,))
          r = g * 16 + i
          for j in range(F // 16):
            rows_v[b, r, pl.ds(j * 16, 16)] = (
                rows_v[b, r, pl.ds(j * 16, 16)] * wb)

      wait_d(bd)
      start_scatter(b, bd)

      @pl.when(c + IL < NCHUNK)
      def _():
        issue_sw(c + IL, (c + IL) % NB)
        issue_d(c + IL, (c + IL) % DB)

    for cc in range(NCHUNK - NB, NCHUNK):
      wait_scatter(cc % NB)
    plsc.subcore_barrier()
    pltpu.sync_copy(acc_sh.at[pl.ds(sid * RPS, RPS)],
                    out_hbm.at[cid, pl.ds(sid * RPS, RPS)])

  return k(x, src, dst, w, zeros)


BLK = 400


def _tc_finish(p0, p1, Wm, b2):
  def body(p0_ref, p1_ref, w_ref, b_ref, o_ref):
    acc = p0_ref[...] + p1_ref[...]
    h = jnp.dot(acc, w_ref[...], preferred_element_type=jnp.float32)
    o_ref[...] = jnp.maximum(h + b_ref[...], 0.0)

  return pl.pallas_call(
      body,
      grid=(N_NODES // BLK,),
      in_specs=[
          pl.BlockSpec((BLK, F), lambda i: (i, 0)),
          pl.BlockSpec((BLK, F), lambda i: (i, 0)),
          pl.BlockSpec((F, F), lambda i: (0, 0)),
          pl.BlockSpec((1, F), lambda i: (0, 0)),
      ],
      out_specs=pl.BlockSpec((BLK, F), lambda i: (i, 0)),
      out_shape=jax.ShapeDtypeStruct((N_NODES, F), jnp.float32),
  )(p0, p1, Wm, b2)


def kernel(x, edge_index, edge_weight, W, b):
  src = edge_index[0].astype(jnp.int32)
  dst = edge_index[1].astype(jnp.int32)
  zeros = jnp.zeros((RPS, F), jnp.float32)
  parts = _sc_aggregate(x, src, dst, edge_weight, zeros)
  return _tc_finish(parts[0, :N_NODES], parts[1, :N_NODES], W, b[None, :])


# speedup vs baseline: 2.9427x; 1.0438x over previous
import functools

import jax
import jax.numpy as jnp
from jax import lax
from jax.experimental import pallas as pl
from jax.experimental.pallas import tpu as pltpu
from jax.experimental.pallas import tpu_sc as plsc

N_NODES = 10000
N_EDGES = 320000
F = 128
NC, NS = 2, 16
NW = NC * NS
EPW = N_EDGES // NW
CH = 80
NCHUNK = EPW // CH
P_NODES = 10240
RPS = P_NODES // NS


def _sc_aggregate(x, src, dst, w, zeros):
  mesh = plsc.VectorSubcoreMesh(
      core_axis_name="c", subcore_axis_name="s", num_cores=NC, num_subcores=NS)

  NB = 4
  DB = 8
  GL = 2
  IL = 4

  @functools.partial(
      pl.kernel,
      out_type=jax.ShapeDtypeStruct((NC, P_NODES, F), jnp.float32),
      mesh=mesh,
      scratch_types=dict(
          src_v=pltpu.VMEM((NB, CH), jnp.int32),
          dst_v=pltpu.VMEM((DB, CH), jnp.int32),
          w_v=pltpu.VMEM((NB, CH), jnp.float32),
          rows_v=pltpu.VMEM((NB, CH, F), jnp.float32),
          acc_sh=pltpu.VMEM_SHARED((P_NODES, F), jnp.float32),
          sem_g=pltpu.SemaphoreType.DMA((NB,)),
          sem_i=pltpu.SemaphoreType.DMA((NB,)),
          sem_d=pltpu.SemaphoreType.DMA((DB,)),
          sem_s=pltpu.SemaphoreType.DMA((NB,)),
      ),
      compiler_params=pltpu.CompilerParams(needs_layout_passes=False),
  )
  def k(x_hbm, src_hbm, dst_hbm, w_hbm, z_hbm, out_hbm,
        src_v, dst_v, w_v, rows_v, acc_sh, sem_g, sem_i, sem_d, sem_s):
    cid = lax.axis_index("c")
    sid = lax.axis_index("s")
    wid = cid * NS + sid
    base = wid * EPW

    def issue_sw(cc, b):
      off = base + cc * CH
      pltpu.async_copy(src_hbm.at[pl.ds(off, CH)], src_v.at[b], sem_i.at[b])
      pltpu.async_copy(w_hbm.at[pl.ds(off, CH)], w_v.at[b], sem_i.at[b])

    def wait_sw(b):
      pltpu.make_async_copy(src_hbm.at[pl.ds(0, CH)], src_v.at[b],
                            sem_i.at[b]).wait()
      pltpu.make_async_copy(w_hbm.at[pl.ds(0, CH)], w_v.at[b],
                            sem_i.at[b]).wait()

    def issue_d(cc, bd):
      off = base + cc * CH
      pltpu.async_copy(dst_hbm.at[pl.ds(off, CH)], dst_v.at[bd], sem_d.at[bd])

    def wait_d(bd):
      pltpu.make_async_copy(dst_hbm.at[pl.ds(0, CH)], dst_v.at[bd],
                            sem_d.at[bd]).wait()

    def start_gather(b):
      pltpu.async_copy(x_hbm.at[src_v.at[b]], rows_v.at[b], sem_g.at[b])

    def wait_gather(b):
      pltpu.make_async_copy(x_hbm.at[src_v.at[b]], rows_v.at[b],
                            sem_g.at[b]).wait()

    def start_scatter(b, bd):
      pltpu.async_copy(rows_v.at[b], acc_sh.at[dst_v.at[bd]], sem_s.at[b],
                       add=True)

    def wait_scatter(b):
      pltpu.make_async_copy(rows_v.at[b], acc_sh.at[dst_v.at[0]],
                            sem_s.at[b]).wait()

    for cc in range(IL):
      issue_sw(cc, cc % NB)
      issue_d(cc, cc % DB)
    pltpu.sync_copy(z_hbm, acc_sh.at[pl.ds(sid * RPS, RPS)])
    plsc.subcore_barrier()
    for cc in range(GL):
      wait_sw(cc % NB)
      start_gather(cc % NB)

    @pl.loop(0, NCHUNK)
    def _chunk(c):
      b = c % NB
      bd = c % DB

      wait_gather(b)

      @pl.when(c + GL < NCHUNK)
      def _():
        bg = (c + GL) % NB
        wait_sw(bg)

        @pl.when(c + GL >= NB)
        def _():
          wait_scatter(bg)

        start_gather(bg)

      iota16 = lax.iota(jnp.int32, 16)
      for g in range(CH // 16):
        wv = w_v[b, pl.ds(g * 16, 16)]
        for i in range(16):
          ws = jnp.max(jnp.where(iota16 == i, wv, -jnp.inf))
          wb = jnp.broadcast_to(ws, (16,))
          r = g * 16 + i
          for j in range(F // 16):
            rows_v[b, r, pl.ds(j * 16, 16)] = (
                rows_v[b, r, pl.ds(j * 16, 16)] * wb)

      wait_d(bd)
      start_scatter(b, bd)

      @pl.when(c + IL < NCHUNK)
      def _():
        issue_sw(c + IL, (c + IL) % NB)
        issue_d(c + IL, (c + IL) % DB)

    for cc in range(NCHUNK - NB, NCHUNK):
      wait_scatter(cc % NB)
    plsc.subcore_barrier()
    pltpu.sync_copy(acc_sh.at[pl.ds(sid * RPS, RPS)],
                    out_hbm.at[cid, pl.ds(sid * RPS, RPS)])

  return k(x, src, dst, w, zeros)


BLK = 400


def _tc_finish(parts, Wm, b2):
  def body(p0_ref, p1_ref, w_ref, b_ref, o_ref):
    acc = p0_ref[0] + p1_ref[0]
    h = jnp.dot(acc, w_ref[...], preferred_element_type=jnp.float32)
    o_ref[...] = jnp.maximum(h + b_ref[...], 0.0)

  return pl.pallas_call(
      body,
      grid=(N_NODES // BLK,),
      in_specs=[
          pl.BlockSpec((1, BLK, F), lambda i: (0, i, 0)),
          pl.BlockSpec((1, BLK, F), lambda i: (1, i, 0)),
          pl.BlockSpec((F, F), lambda i: (0, 0)),
          pl.BlockSpec((1, F), lambda i: (0, 0)),
      ],
      out_specs=pl.BlockSpec((BLK, F), lambda i: (i, 0)),
      out_shape=jax.ShapeDtypeStruct((N_NODES, F), jnp.float32),
  )(parts, parts, Wm, b2)


def kernel(x, edge_index, edge_weight, W, b):
  src = edge_index[0].astype(jnp.int32)
  dst = edge_index[1].astype(jnp.int32)
  zeros = jnp.zeros((RPS, F), jnp.float32)
  parts = _sc_aggregate(x, src, dst, edge_weight, zeros)
  return _tc_finish(parts, W, b[None, :])
